# baseline (speedup 1.0000x reference)
import functools

import jax
import jax.numpy as jnp
from jax import lax
from jax.experimental import pallas as pl
from jax.experimental.pallas import tpu as pltpu
from jax.experimental.pallas import tpu_sc as plsc

B = 4096
L = 200
D_IN = 64
STATION_DIM = 16
SEASON_DIM = 4
N_SEASONS = 4
D_OUT = D_IN + STATION_DIM + SEASON_DIM

_NC = 2
_NS = 16
_NW = _NC * _NS
_B_PER_W = B // _NW
_G = 2
_NSTEP = _B_PER_W // _G


def _sc_assemble(x, station_ids, season_ids, station_table, season_table):
    mesh = plsc.VectorSubcoreMesh(core_axis_name="c", subcore_axis_name="s")

    @functools.partial(
        pl.kernel,
        mesh=mesh,
        out_type=jax.ShapeDtypeStruct((B, L, D_OUT), jnp.float32),
        scratch_types=[
            pltpu.VMEM((_B_PER_W,), jnp.int32),
            pltpu.VMEM((_B_PER_W, STATION_DIM), jnp.float32),
            pltpu.VMEM((2, _G, L, STATION_DIM), jnp.float32),
            pltpu.VMEM((2, _G, L, SEASON_DIM), jnp.float32),
            pltpu.VMEM((N_SEASONS, SEASON_DIM), jnp.float32),
            pltpu.VMEM((_B_PER_W,), jnp.int32),
            pltpu.SemaphoreType.DMA,
            pltpu.SemaphoreType.DMA,
            pltpu.SemaphoreType.DMA,
            pltpu.SemaphoreType.DMA,
        ],
        compiler_params=pltpu.CompilerParams(use_tc_tiling_on_sc=False,
                                             needs_layout_passes=False),
    )
    def k(x_hbm, sid_hbm, seid_hbm, table_hbm, stab_hbm, out_hbm,
          idx_v, st_rows, st_bc, se_bc, stab_v, sed_v,
          sem, sem_x, sem_st, sem_se):
        wid = lax.axis_index("s") * _NC + lax.axis_index("c")
        base = wid * _B_PER_W
        pltpu.make_async_copy(
            x_hbm.at[pl.ds(base, _B_PER_W)],
            out_hbm.at[pl.ds(base, _B_PER_W), :, 0:D_IN], sem_x).start()
        pltpu.sync_copy(sid_hbm.at[pl.ds(base, _B_PER_W)], idx_v)
        pltpu.async_copy(table_hbm.at[idx_v], st_rows, sem).wait()
        pltpu.sync_copy(seid_hbm.at[pl.ds(base, _B_PER_W)], sed_v)
        pltpu.sync_copy(stab_hbm, stab_v)

        lanes = lax.iota(jnp.int32, 16)

        def body(m, carry):
            b = base + m * _G
            pp = m & 1
            @pl.when(m >= 2)
            def _():
                pltpu.make_async_copy(
                    st_bc.at[pp],
                    out_hbm.at[pl.ds(b, _G), :, D_IN:D_IN + STATION_DIM],
                    sem_st).wait()
                pltpu.make_async_copy(
                    se_bc.at[pp],
                    out_hbm.at[pl.ds(b, _G), :, D_IN + STATION_DIM:D_OUT],
                    sem_se).wait()
            for r in range(_G):
                j = m * _G + r
                stv = st_rows[j, :]
                def fill_l(mm, c0, _r=r, _stv=stv):
                    ll = mm * 8
                    for i in range(8):
                        st_bc[pp, _r, ll + i, :] = _stv
                    return c0
                lax.fori_loop(0, L // 8, fill_l, 0)
                sid_splat = plsc.load_gather(
                    sed_v, [jnp.full((16,), j, jnp.int32)])
                pat = plsc.load_gather(stab_v, [sid_splat, lanes & 3])
                def fill_t(tt, c0, _r=r, _pat=pat):
                    for i in range(10):
                        flat = (tt * 10 + i) * 16 + lanes
                        plsc.store_scatter(
                            se_bc,
                            [jnp.full((16,), pp, jnp.int32),
                             jnp.full((16,), _r, jnp.int32),
                             flat >> 2, flat & 3], _pat)
                    return c0
                lax.fori_loop(0, L * SEASON_DIM // 160, fill_t, 0)
            pltpu.make_async_copy(
                st_bc.at[pp],
                out_hbm.at[pl.ds(b, _G), :, D_IN:D_IN + STATION_DIM],
                sem_st).start()
            pltpu.make_async_copy(
                se_bc.at[pp],
                out_hbm.at[pl.ds(b, _G), :, D_IN + STATION_DIM:D_OUT],
                sem_se).start()
            return carry
        lax.fori_loop(0, _NSTEP, body, 0)

        def drain(i, carry):
            pltpu.make_async_copy(
                st_bc.at[0],
                out_hbm.at[pl.ds(base, _G), :, D_IN:D_IN + STATION_DIM],
                sem_st).wait()
            pltpu.make_async_copy(
                se_bc.at[0],
                out_hbm.at[pl.ds(base, _G), :, D_IN + STATION_DIM:D_OUT],
                sem_se).wait()
            return carry
        lax.fori_loop(0, 2, drain, 0)
        pltpu.make_async_copy(
            x_hbm.at[pl.ds(base, _B_PER_W)],
            out_hbm.at[pl.ds(base, _B_PER_W), :, 0:D_IN], sem_x).wait()

    return k(x, station_ids, season_ids, station_table, season_table)


def kernel(x, station_ids, season_ids, station_table, season_table):
    return _sc_assemble(x, station_ids, season_ids, station_table,
                        season_table)


# speedup vs baseline: 9.0739x; 9.0739x over previous
import functools

import jax
import jax.numpy as jnp
from jax import lax
from jax.experimental import pallas as pl
from jax.experimental.pallas import tpu as pltpu
from jax.experimental.pallas import tpu_sc as plsc

B = 4096
L = 200
D_IN = 64
STATION_DIM = 16
SEASON_DIM = 4
N_SEASONS = 4
D_OUT = D_IN + STATION_DIM + SEASON_DIM

_NC = 2
_NS = 16
_NW = _NC * _NS
_B_PER_W = B // _NW


def _sc_station_gather(station_table, station_ids):
    mesh = plsc.VectorSubcoreMesh(core_axis_name="c", subcore_axis_name="s")

    @functools.partial(
        pl.kernel,
        mesh=mesh,
        out_type=jax.ShapeDtypeStruct((B, STATION_DIM), jnp.float32),
        scratch_types=[
            pltpu.VMEM((_B_PER_W,), jnp.int32),
            pltpu.VMEM((_B_PER_W, STATION_DIM), jnp.float32),
            pltpu.SemaphoreType.DMA,
        ],
        compiler_params=pltpu.CompilerParams(use_tc_tiling_on_sc=False),
    )
    def k(table_hbm, idx_hbm, out_hbm, idx_v, rows_v, sem):
        wid = lax.axis_index("s") * _NC + lax.axis_index("c")
        base = wid * _B_PER_W
        pltpu.sync_copy(idx_hbm.at[pl.ds(base, _B_PER_W)], idx_v)
        pltpu.async_copy(table_hbm.at[idx_v], rows_v, sem).wait()
        pltpu.sync_copy(rows_v, out_hbm.at[pl.ds(base, _B_PER_W)])

    return k(station_table, station_ids)


_R = 128
_NB = B // _R


def _tc_concat_body(x_ref, st_ref, sid_ref, stab_ref, out_ref):
    xb = x_ref[...]
    st = st_ref[...]
    sid = sid_ref[0]
    se = jnp.zeros((_R, SEASON_DIM), dtype=jnp.float32)
    for k in range(N_SEASONS):
        row = stab_ref[k:k + 1, :]
        se = se + jnp.where(sid == k, 1.0, 0.0) * row
    out_ref[:, :, 0:D_IN] = xb
    out_ref[:, :, D_IN:D_IN + STATION_DIM] = jnp.broadcast_to(
        st[:, None, :], (_R, L, STATION_DIM))
    out_ref[:, :, D_IN + STATION_DIM:D_OUT] = jnp.broadcast_to(
        se[:, None, :], (_R, L, SEASON_DIM))


def _tc_concat(x, station_embed, season_ids, season_table):
    sid3 = season_ids.reshape(_NB, _R, 1)
    return pl.pallas_call(
        _tc_concat_body,
        grid=(_NB,),
        in_specs=[
            pl.BlockSpec((_R, L, D_IN), lambda i: (i, 0, 0)),
            pl.BlockSpec((_R, STATION_DIM), lambda i: (i, 0)),
            pl.BlockSpec((1, _R, 1), lambda i: (i, 0, 0)),
            pl.BlockSpec((N_SEASONS, SEASON_DIM), lambda i: (0, 0)),
        ],
        out_specs=pl.BlockSpec((_R, L, D_OUT), lambda i: (i, 0, 0)),
        out_shape=jax.ShapeDtypeStruct((B, L, D_OUT), jnp.float32),
        compiler_params=pltpu.CompilerParams(
            dimension_semantics=("parallel",)),
    )(x, station_embed, sid3, season_table)


def kernel(x, station_ids, season_ids, station_table, season_table):
    station_embed = _sc_station_gather(station_table, station_ids)
    return _tc_concat(x, station_embed, season_ids, season_table)
